# baseline (speedup 1.0000x reference)
import jax
import jax.numpy as jnp
from jax import lax
from jax.experimental import pallas as pl
from jax.experimental.pallas import tpu as pltpu
from jax.experimental.pallas import tpu_sc as plsc

B, T, D = 16, 1024, 256
L = 8192
NC, NS = 2, 16
NW = NC * NS
PW = B * L // NW
CHUNK = 128
NCHUNK = PW // CHUNK
NBUF = 3
VL = 16
ZROW = B * T


def _body(x_hbm, dur_hbm, out_hbm, mel_hbm,
          dur_v, a_v, idx_v, buf0, buf1, buf2, mel_v,
          gsem, wsem):
    cid = lax.axis_index("c")
    sid = lax.axis_index("s")
    wid = sid * NC + cid
    b = wid // 2
    half = wid % 2
    p0 = half * (L // 2)

    pltpu.sync_copy(dur_hbm.at[b], dur_v.at[pl.ds(0, T)])
    dur_v[pl.ds(T, VL)] = jnp.zeros((VL,), jnp.int32)

    def zero_body(i, _):
        a_v[pl.ds(i * VL, VL)] = jnp.zeros((VL,), jnp.int32)
        return 0
    lax.fori_loop(0, PW // VL, zero_body, 0)

    lane = jnp.arange(VL, dtype=jnp.int32)

    def scat_body(j, carry_base):
        carry, base = carry_base
        v = dur_v[pl.ds(j * VL, VL)]
        s = plsc.cumsum(v) + carry
        i_vec = lane + j * VL
        d_next = dur_v[pl.ds(j * VL + 1, VL)]
        run_end = (d_next != 0) | (i_vec == T - 1)
        local = s - p0
        m = run_end & (local >= 0) & (local < PW)
        plsc.store_scatter(a_v, (jnp.where(m, local, 0),), i_vec + 1, mask=m)
        base = base + jnp.sum(jnp.where(s < p0, 1, 0).astype(jnp.int32))
        return (jnp.max(s), base)

    total, base = lax.fori_loop(
        0, T // VL, scat_body, (jnp.int32(0), jnp.int32(0)))

    rowbase = b * T

    def idx_body(i, carry):
        v = a_v[pl.ds(i * VL, VL)]
        s = jnp.maximum(plsc.cummax(v), carry)
        g = jnp.where(s >= T, ZROW, s + rowbase)
        idx_v[pl.ds(i * VL, VL)] = g
        return jnp.max(s)

    lax.fori_loop(0, PW // VL, idx_body, base)

    @pl.when(half == 0)
    def _():
        mel_v[...] = jnp.full((VL,), total, jnp.int32)
        pltpu.sync_copy(mel_v, mel_hbm.at[b])

    row0 = wid * PW
    bufs = [buf0, buf1, buf2]

    def gather(c, buf):
        return pltpu.async_copy(
            x_hbm.at[idx_v.at[pl.ds(c * CHUNK, CHUNK)]], buf, gsem)

    def write(c, buf):
        return pltpu.async_copy(
            buf, out_hbm.at[pl.ds(row0 + c * CHUNK, CHUNK)], wsem)

    gdesc = [None] * NCHUNK
    wdesc = [None] * NCHUNK
    for c in range(NCHUNK):
        if c >= NBUF:
            wdesc[c - NBUF].wait()
        gdesc[c] = gather(c, bufs[c % NBUF])
        if c >= NBUF - 1:
            cc = c - (NBUF - 1)
            gdesc[cc].wait()
            wdesc[cc] = write(cc, bufs[cc % NBUF])
    for cc in range(NCHUNK - NBUF + 1, NCHUNK):
        gdesc[cc].wait()
        wdesc[cc] = write(cc, bufs[cc % NBUF])
    for cc in range(NCHUNK - NBUF, NCHUNK):
        wdesc[cc].wait()


import functools


@functools.cache
def _regulate():
    return pl.kernel(
        _body,
        out_type=[
            jax.ShapeDtypeStruct((B * L, D), jnp.float32),
            jax.ShapeDtypeStruct((B, VL), jnp.int32),
        ],
        mesh=plsc.VectorSubcoreMesh(core_axis_name="c", subcore_axis_name="s",
                                    num_cores=NC, num_subcores=NS),
        compiler_params=pltpu.CompilerParams(needs_layout_passes=False),
        scratch_types=[
            pltpu.VMEM((T + VL,), jnp.int32),
            pltpu.VMEM((PW,), jnp.int32),
            pltpu.VMEM((PW,), jnp.int32),
            pltpu.VMEM((CHUNK, D), jnp.float32),
            pltpu.VMEM((CHUNK, D), jnp.float32),
            pltpu.VMEM((CHUNK, D), jnp.float32),
            pltpu.VMEM((VL,), jnp.int32),
            pltpu.SemaphoreType.DMA,
            pltpu.SemaphoreType.DMA,
        ],
    )


def kernel(x, duration, max_len):
    x_pad = jnp.concatenate(
        [x.reshape(B * T, D), jnp.zeros((8, D), x.dtype)], axis=0)
    out_flat, mel2 = _regulate()(x_pad, duration.astype(jnp.int32))
    return out_flat.reshape(B, L, D), mel2[:, 0]


# speedup vs baseline: 5.6003x; 5.6003x over previous
import functools

import jax
import jax.numpy as jnp
from jax import lax
from jax.experimental import pallas as pl
from jax.experimental.pallas import tpu as pltpu
from jax.experimental.pallas import tpu_sc as plsc

B, T, D = 16, 1024, 256
L = 8192
NC, NS = 2, 16
NW = NC * NS
PW = B * L // NW
CHUNK = 128
NCHUNK = PW // CHUNK
SROWS = CHUNK + 8
VL = 16
DV = D // VL


def _body(x_hbm, dur_hbm, out_hbm, mel_hbm,
          dur_v, a_v, idx_v, sbuf, ob0, ob1, mel_v,
          gsem, wsem):
    cid = lax.axis_index("c")
    sid = lax.axis_index("s")
    wid = sid * NC + cid
    b = wid // 2
    half = wid % 2
    p0 = half * (L // 2)
    lane = jnp.arange(VL, dtype=jnp.int32)
    obufs = [ob0, ob1]

    pltpu.sync_copy(dur_hbm.at[b], dur_v.at[pl.ds(0, T)])
    dur_v[pl.ds(T, VL)] = jnp.zeros((VL,), jnp.int32)

    def zero_body(i, _):
        a_v[pl.ds(i * VL, VL)] = jnp.zeros((VL,), jnp.int32)
        return 0
    lax.fori_loop(0, PW // VL, zero_body, 0)

    def scat_body(j, carry_base):
        carry, base = carry_base
        v = dur_v[pl.ds(j * VL, VL)]
        s = plsc.cumsum(v) + carry
        i_vec = lane + j * VL
        d_next = dur_v[pl.ds(j * VL + 1, VL)]
        run_end = (d_next != 0) | (i_vec == T - 1)
        local = s - p0
        m = run_end & (local >= 0) & (local < PW)
        plsc.store_scatter(a_v, (jnp.where(m, local, 0),), i_vec + 1, mask=m)
        base = base + jnp.sum(jnp.where(s < p0, 1, 0).astype(jnp.int32))
        return (jnp.max(s), base)

    total, base = lax.fori_loop(
        0, T // VL, scat_body, (jnp.int32(0), jnp.int32(0)))

    rowbase = b * T

    def idx_body(i, carry):
        v = a_v[pl.ds(i * VL, VL)]
        s = jnp.maximum(plsc.cummax(v), carry)
        idx_v[pl.ds(i * VL, VL)] = jnp.minimum(s, T - 1) + rowbase
        return jnp.max(s)

    lax.fori_loop(0, PW // VL, idx_body, base)
    idx_v[pl.ds(PW, VL)] = jnp.full((VL,), rowbase + T - 1, jnp.int32)

    @pl.when(half == 0)
    def _():
        mel_v[...] = jnp.full((VL,), total, jnp.int32)
        pltpu.sync_copy(mel_v, mel_hbm.at[b])

    n_valid = jnp.clip(total - p0, 0, PW)
    row0 = wid * PW

    def src_row(p):
        return jnp.min(idx_v[pl.ds(p, VL)])

    wdesc = [None] * NCHUNK
    for c in range(NCHUNK):
        if c >= 2:
            wdesc[c - 2].wait()
        ob = obufs[c % 2]
        c_lo = c * CHUNK
        r = jnp.clip(n_valid - c_lo, 0, CHUNK)
        lo_g = src_row(c_lo)
        hi_g = src_row(c_lo + jnp.maximum(r - 1, 0))
        span = hi_g - lo_g + 1
        start = pl.multiple_of(
            jnp.minimum((lo_g // 8) * 8, B * T - SROWS), 8)

        @pl.when((r > 0) & (span <= CHUNK))
        def _():
            pltpu.async_copy(
                x_hbm.at[pl.ds(start * D, SROWS * D)], sbuf, gsem).wait()

            def expand(p, _):
                so = jnp.clip(src_row(c_lo + p) - start, 0, SROWS - 1) * D
                po = p * D
                for d in range(DV):
                    ob[pl.ds(po + d * VL, VL)] = sbuf[pl.ds(so + d * VL, VL)]
                return 0
            lax.fori_loop(0, r, expand, 0)

        @pl.when((r > 0) & (span > CHUNK))
        def _():
            def row_copy(p, _):
                g = src_row(c_lo + p)
                pltpu.sync_copy(x_hbm.at[pl.ds(g * D, D)],
                                ob.at[pl.ds(p * D, D)])
                return 0
            lax.fori_loop(0, r, row_copy, 0)

        def zrow(p, _):
            po = p * D
            for d in range(DV):
                ob[pl.ds(po + d * VL, VL)] = jnp.zeros((VL,), jnp.float32)
            return 0
        lax.fori_loop(r, CHUNK, zrow, 0)

        wdesc[c] = pltpu.async_copy(
            ob, out_hbm.at[pl.ds((row0 + c_lo) * D, CHUNK * D)], wsem)

    wdesc[NCHUNK - 2].wait()
    wdesc[NCHUNK - 1].wait()


@functools.cache
def _regulate():
    return pl.kernel(
        _body,
        out_type=[
            jax.ShapeDtypeStruct((B * L * D,), jnp.float32),
            jax.ShapeDtypeStruct((B, VL), jnp.int32),
        ],
        mesh=plsc.VectorSubcoreMesh(core_axis_name="c", subcore_axis_name="s",
                                    num_cores=NC, num_subcores=NS),
        compiler_params=pltpu.CompilerParams(needs_layout_passes=False),
        scratch_types=[
            pltpu.VMEM((T + VL,), jnp.int32),
            pltpu.VMEM((PW,), jnp.int32),
            pltpu.VMEM((PW + VL,), jnp.int32),
            pltpu.VMEM((SROWS * D,), jnp.float32),
            pltpu.VMEM((CHUNK * D,), jnp.float32),
            pltpu.VMEM((CHUNK * D,), jnp.float32),
            pltpu.VMEM((VL,), jnp.int32),
            pltpu.SemaphoreType.DMA,
            pltpu.SemaphoreType.DMA,
        ],
    )


def kernel(x, duration, max_len):
    out_flat, mel2 = _regulate()(x.reshape(B * T * D),
                                 duration.astype(jnp.int32))
    return out_flat.reshape(B, L, D), mel2[:, 0]
